# baseline (speedup 1.0000x reference)
import functools

import jax
import jax.numpy as jnp
from jax import lax
from jax.experimental import pallas as pl
from jax.experimental.pallas import tpu as pltpu
from jax.experimental.pallas import tpu_sc as plsc

_BLK_S = 1024

_SC_NC = 2
_SC_NS = 16
_SC_NW = _SC_NC * _SC_NS
_SC_LANES = 16
_SC_CHUNK_WORDS = 16 * 2048


def _tc_add_kernel(x_ref, t_ref, o_ref):
    o_ref[...] = x_ref[...] + t_ref[...]


def _tc_add(x, table):
    b, s, d = x.shape
    grid = (s // _BLK_S, b)
    return pl.pallas_call(
        _tc_add_kernel,
        grid=grid,
        in_specs=[
            pl.BlockSpec((1, _BLK_S, d), lambda j, i: (i, j, 0)),
            pl.BlockSpec((_BLK_S, d), lambda j, i: (j, 0)),
        ],
        out_specs=pl.BlockSpec((1, _BLK_S, d), lambda j, i: (i, j, 0)),
        out_shape=jax.ShapeDtypeStruct((b, s, d), x.dtype),
    )(x, table)


def _sc_add(x_flat, t_flat):
    (n,) = x_flat.shape
    words_per_w = n // _SC_NW
    n_chunks = words_per_w // _SC_CHUNK_WORDS
    mesh = plsc.VectorSubcoreMesh(core_axis_name="c", subcore_axis_name="s")

    @functools.partial(
        pl.kernel,
        out_type=jax.ShapeDtypeStruct((n,), jnp.float32),
        mesh=mesh,
        scratch_types=[
            pltpu.VMEM((_SC_CHUNK_WORDS,), jnp.float32),
            pltpu.VMEM((_SC_CHUNK_WORDS,), jnp.float32),
        ],
    )
    def _sc_kernel(x_hbm, t_hbm, o_hbm, bufx, buft):
        wid = lax.axis_index("s") * _SC_NC + lax.axis_index("c")
        base = wid * words_per_w
        for c in range(n_chunks):
            off = base + c * _SC_CHUNK_WORDS
            pltpu.sync_copy(x_hbm.at[pl.ds(off, _SC_CHUNK_WORDS)], bufx)
            pltpu.sync_copy(t_hbm.at[pl.ds(off, _SC_CHUNK_WORDS)], buft)

            def body(j, carry):
                sl = pl.ds(j * _SC_LANES, _SC_LANES)
                plsc.addupdate(bufx.at[sl], buft[sl])
                return carry

            lax.fori_loop(0, _SC_CHUNK_WORDS // _SC_LANES, body, 0)
            pltpu.sync_copy(bufx, o_hbm.at[pl.ds(off, _SC_CHUNK_WORDS)])

    return _sc_kernel(x_flat, t_flat)


def kernel(x, table):
    b, s, d = x.shape
    table_s = table[:s]
    out_tc = _tc_add(x[: b - 1], table_s)
    out_sc = _sc_add(x[b - 1].reshape(-1), table_s.reshape(-1))
    return jnp.concatenate([out_tc, out_sc.reshape(1, s, d)], axis=0)


# speedup vs baseline: 4.2382x; 4.2382x over previous
import jax
import jax.numpy as jnp
from jax.experimental import pallas as pl
from jax.experimental.pallas import tpu as pltpu

_BLK_S = 512
_NBUF = 4


def _add_block(x_ref, t_ref, o_ref):
    o_ref[...] = x_ref[...] + t_ref[...]


def kernel(x, table):
    b, s, d = x.shape
    table_s = table[:s]

    def outer(x_hbm, t_hbm, o_hbm):
        pipeline = pltpu.emit_pipeline(
            _add_block,
            grid=(s // _BLK_S, b),
            in_specs=[
                pl.BlockSpec((1, _BLK_S, d), lambda j, i: (i, j, 0),
                             pipeline_mode=pl.Buffered(buffer_count=_NBUF)),
                pl.BlockSpec((_BLK_S, d), lambda j, i: (j, 0)),
            ],
            out_specs=[
                pl.BlockSpec((1, _BLK_S, d), lambda j, i: (i, j, 0)),
            ],
        )
        pipeline(x_hbm, t_hbm, o_hbm)

    return pl.pallas_call(
        outer,
        in_specs=[
            pl.BlockSpec(memory_space=pl.ANY),
            pl.BlockSpec(memory_space=pl.ANY),
        ],
        out_specs=pl.BlockSpec(memory_space=pl.ANY),
        out_shape=jax.ShapeDtypeStruct((b, s, d), x.dtype),
    )(x, table_s)


# speedup vs baseline: 4.4719x; 1.0551x over previous
import jax
import jax.numpy as jnp
from jax.experimental import pallas as pl
from jax.experimental.pallas import tpu as pltpu

_BLK_S = 1024
_NBUF = 3


def _add_block(x_ref, t_ref, o_ref):
    o_ref[...] = x_ref[...] + t_ref[...]


def kernel(x, table):
    b, s, d = x.shape
    table_s = table[:s]

    def outer(x_hbm, t_hbm, o_hbm):
        pipeline = pltpu.emit_pipeline(
            _add_block,
            grid=(s // _BLK_S, b),
            in_specs=[
                pl.BlockSpec((1, _BLK_S, d), lambda j, i: (i, j, 0),
                             pipeline_mode=pl.Buffered(buffer_count=_NBUF)),
                pl.BlockSpec((_BLK_S, d), lambda j, i: (j, 0)),
            ],
            out_specs=[
                pl.BlockSpec((1, _BLK_S, d), lambda j, i: (i, j, 0)),
            ],
        )
        pipeline(x_hbm, t_hbm, o_hbm)

    return pl.pallas_call(
        outer,
        in_specs=[
            pl.BlockSpec(memory_space=pl.ANY),
            pl.BlockSpec(memory_space=pl.ANY),
        ],
        out_specs=pl.BlockSpec(memory_space=pl.ANY),
        out_shape=jax.ShapeDtypeStruct((b, s, d), x.dtype),
    )(x, table_s)
